# Initial kernel scaffold; baseline (speedup 1.0000x reference)
import jax
import jax.numpy as jnp
from jax.experimental import pallas as pl


def kernel(inp, edge_index, edge_val, basis_weights, basis_coeff, bias):
    raise NotImplementedError("write your pallas kernel here")


# speedup vs baseline: 1.7105x; 1.7105x over previous
import functools

import jax
import jax.numpy as jnp
from jax import lax
from jax.experimental import pallas as pl
from jax.experimental.pallas import tpu as pltpu
from jax.experimental.pallas import tpu_sc as plsc

_N = 10000
_E = 80000
_R = 4
_D = 128
_NB = 2

_NC = 2
_NS = 16
_LANES = 16

_CHUNK = 80
_EDGES = _R * _E
_EPT = _EDGES // _NS
_NCHUNK = _EPT // _CHUNK
_NPAD = 10240
_ROWS_PT = _NPAD // _NS
_ZROWS = 128


def _sc_body(inp_hbm, src_hbm, dst_hbm, val_hbm, coeff_hbm, acc_hbm,
             acc_sp, src_v, dst_v, val_v, rows_v, zbuf_v, coeff_v, sem):
    c = lax.axis_index("c")
    s = lax.axis_index("s")

    def _zrow(i, carry):
        for j in range(_D // _LANES):
            zbuf_v[i, pl.ds(j * _LANES, _LANES)] = jnp.zeros((_LANES,), jnp.float32)
        return carry
    lax.fori_loop(0, _ZROWS, _zrow, 0)
    for t in range(_ROWS_PT // _ZROWS):
        pltpu.sync_copy(zbuf_v, acc_sp.at[pl.ds(s * _ROWS_PT + t * _ZROWS, _ZROWS)])
    pltpu.sync_copy(coeff_hbm, coeff_v)
    plsc.subcore_barrier()

    rel = s // (_NS // _R)
    cv = coeff_v[...]
    want = jnp.full((_LANES,), rel * _NB + c, jnp.int32)
    cvec = cv.at[want].get(mode="promise_in_bounds")
    base_e = s * _EPT

    def _chunk(k, carry):
        off = base_e + k * _CHUNK
        pltpu.sync_copy(src_hbm.at[pl.ds(off, _CHUNK)], src_v)
        pltpu.sync_copy(dst_hbm.at[pl.ds(off, _CHUNK)], dst_v)
        pltpu.sync_copy(val_hbm.at[pl.ds(off, _CHUNK)], val_v)
        pltpu.async_copy(inp_hbm.at[src_v], rows_v, sem).wait()
        for j in range(_CHUNK // _LANES):
            val_v[pl.ds(j * _LANES, _LANES)] = (
                val_v[pl.ds(j * _LANES, _LANES)] * cvec)

        def _egroup(g, ecarry):
            w16 = val_v[pl.ds(g * _LANES, _LANES)]
            for l in range(_LANES):
                w = w16[l]
                e = g * _LANES + l
                for j in range(_D // _LANES):
                    sl = pl.ds(j * _LANES, _LANES)
                    rows_v[e, sl] = rows_v[e, sl] * w
            return ecarry
        lax.fori_loop(0, _CHUNK // _LANES, _egroup, 0)
        pltpu.sync_copy(rows_v, acc_sp.at[dst_v], add=True)
        return carry
    lax.fori_loop(0, _NCHUNK, _chunk, 0)

    plsc.subcore_barrier()
    sl = pl.ds(s * _ROWS_PT, _ROWS_PT)
    pltpu.sync_copy(acc_sp.at[sl], acc_hbm.at[c, sl])


@jax.jit
def _sc_call(inp, src, dst, val, coeff_flat):
    mesh = plsc.VectorSubcoreMesh(core_axis_name="c", subcore_axis_name="s",
                                  num_cores=_NC, num_subcores=_NS)
    return pl.kernel(
        _sc_body,
        out_type=jax.ShapeDtypeStruct((_NB, _NPAD, _D), jnp.float32),
        mesh=mesh,
        scratch_types=[
            pltpu.VMEM_SHARED((_NPAD, _D), jnp.float32),
            pltpu.VMEM((_CHUNK,), jnp.int32),
            pltpu.VMEM((_CHUNK,), jnp.int32),
            pltpu.VMEM((_CHUNK,), jnp.float32),
            pltpu.VMEM((_CHUNK, _D), jnp.float32),
            pltpu.VMEM((_ZROWS, _D), jnp.float32),
            pltpu.VMEM((_LANES,), jnp.float32),
            pltpu.SemaphoreType.DMA,
        ],
    )(inp, src, dst, val, coeff_flat)


_BLK = 2000


def _tc_body(acc_ref, bw_ref, bias_ref, out_ref):
    a0 = acc_ref[0]
    a1 = acc_ref[1]
    out = jnp.dot(a0, bw_ref[0], preferred_element_type=jnp.float32)
    out = out + jnp.dot(a1, bw_ref[1], preferred_element_type=jnp.float32)
    out_ref[...] = out + jnp.sum(bias_ref[...], axis=0)[None, :]


@jax.jit
def _tc_call(acc, basis_weights, bias):
    return pl.pallas_call(
        _tc_body,
        out_shape=jax.ShapeDtypeStruct((_N, _D), jnp.float32),
        grid=(_N // _BLK,),
        in_specs=[
            pl.BlockSpec((_NB, _BLK, _D), lambda i: (0, i, 0)),
            pl.BlockSpec((_NB, _D, _D), lambda i: (0, 0, 0)),
            pl.BlockSpec((_R, _D), lambda i: (0, 0)),
        ],
        out_specs=pl.BlockSpec((_BLK, _D), lambda i: (i, 0)),
    )(acc, basis_weights, bias)


def kernel(inp, edge_index, edge_val, basis_weights, basis_coeff, bias):
    dst = edge_index[:, 0, :].reshape(-1)
    src = edge_index[:, 1, :].reshape(-1)
    val = edge_val.reshape(-1)
    coeff_flat = jnp.zeros((_LANES,), jnp.float32).at[: _R * _NB].set(
        basis_coeff.reshape(-1))
    acc = _sc_call(inp, src, dst, val, coeff_flat)
    return _tc_call(acc, basis_weights, bias)
